# Initial kernel scaffold; baseline (speedup 1.0000x reference)
import functools

import jax
import jax.numpy as jnp
from jax import lax
from jax.experimental import pallas as pl
from jax.experimental.pallas import tpu as pltpu
from jax.experimental.pallas import tpu_sc as plsc

N = 10000
E = 320000
D = 128
ED = 16
NPAD = 10240
NCORES = 2
NSUB = 16
NW = NCORES * NSUB
EPW = E // NW
CHUNK = 80
NCHUNK = EPW // CHUNK
ROWS_PER_TILE = NPAD // NSUB

_f32 = jnp.float32


def _prelude_body(x_ref, wfc_ref, wrel_ref, wattn_ref, z_ref, st_ref):
    z = lax.dot_general(x_ref[...], wfc_ref[...], (((1,), (1,)), ((), ())),
                        precision=lax.Precision.HIGHEST,
                        preferred_element_type=_f32)
    z_ref[...] = z
    wa = wattn_ref[...]
    w1 = wa[:, :D]
    w2 = wa[:, D:]
    u = lax.dot_general(w2, wrel_ref[...], (((1,), (1,)), ((), ())),
                        precision=lax.Precision.HIGHEST,
                        preferred_element_type=_f32)
    uz = u[:, ED:]
    s = lax.dot_general(z, w1, (((1,), (1,)), ((), ())),
                        precision=lax.Precision.HIGHEST,
                        preferred_element_type=_f32)
    t = lax.dot_general(z, uz, (((1,), (1,)), ((), ())),
                        precision=lax.Precision.HIGHEST,
                        preferred_element_type=_f32)
    st_ref[...] = jnp.concatenate([s, t, jnp.zeros((N, 6), _f32)], axis=1)


_prelude = pl.pallas_call(
    _prelude_body,
    out_shape=[jax.ShapeDtypeStruct((N, D), _f32),
               jax.ShapeDtypeStruct((N, 8), _f32)],
)


EBLK = 2000
NEB = E // EBLK


def _q_body(ea_ref, wrel_ref, wattn_ref, brel_ref, q_ref):
    wa = wattn_ref[...]
    w2 = wa[:, D:]
    u = lax.dot_general(w2, wrel_ref[...], (((1,), (1,)), ((), ())),
                        precision=lax.Precision.HIGHEST,
                        preferred_element_type=_f32)
    ue = u[:, :ED]
    c0 = lax.dot_general(brel_ref[...], w2, (((1,), (1,)), ((), ())),
                         precision=lax.Precision.HIGHEST,
                         preferred_element_type=_f32)
    q = lax.dot_general(ue, ea_ref[0], (((1,), (1,)), ((), ())),
                        precision=lax.Precision.HIGHEST,
                        preferred_element_type=_f32)
    q_ref[...] = (q + c0).reshape(1, 1, EBLK)


_edgeq = pl.pallas_call(
    _q_body,
    grid=(NEB,),
    in_specs=[
        pl.BlockSpec((1, EBLK, ED), lambda i: (i, 0, 0)),
        pl.BlockSpec((D, D + ED), lambda i: (0, 0)),
        pl.BlockSpec((1, 2 * D), lambda i: (0, 0)),
        pl.BlockSpec((1, D), lambda i: (0, 0)),
    ],
    out_specs=pl.BlockSpec((1, 1, EBLK), lambda i: (i, 0, 0)),
    out_shape=jax.ShapeDtypeStruct((NEB, 1, EBLK), _f32),
)


def _sc_body(src_hbm, dst_hbm, s_hbm, t_hbm, q_hbm, z_hbm,
             hp_hbm, dp_hbm,
             h_sh, d_sh,
             idx_s, idx_d, sv, tv, qv, gv, rows, zrow, dz):
    c = lax.axis_index("c")
    sid = lax.axis_index("s")

    zeros16 = jnp.zeros((16,), _f32)

    def _zb(i, carry):
        for j in range(8):
            zrow[i, pl.ds(j * 16, 16)] = zeros16
        return carry

    lax.fori_loop(0, 128, _zb, 0)

    def _zd(i, carry):
        dz[pl.ds(i * 16, 16)] = zeros16
        return carry

    lax.fori_loop(0, ROWS_PER_TILE // 16, _zd, 0)

    r0 = sid * ROWS_PER_TILE
    for k in range(ROWS_PER_TILE // 128):
        pltpu.sync_copy(zrow, h_sh.at[pl.ds(r0 + k * 128, 128)])
    pltpu.sync_copy(dz, d_sh.at[pl.ds(r0, ROWS_PER_TILE)])
    plsc.subcore_barrier()

    base = (c * NSUB + sid) * EPW

    def _chunk(i, carry):
        off = base + i * CHUNK
        pltpu.sync_copy(src_hbm.at[pl.ds(off, CHUNK)], idx_s)
        pltpu.sync_copy(dst_hbm.at[pl.ds(off, CHUNK)], idx_d)
        pltpu.sync_copy(q_hbm.at[pl.ds(off, CHUNK)], qv)
        pltpu.sync_copy(s_hbm.at[idx_s], sv)
        pltpu.sync_copy(t_hbm.at[idx_d], tv)
        pltpu.sync_copy(z_hbm.at[idx_s], rows)
        for j in range(CHUNK // 16):
            sl = pl.ds(j * 16, 16)
            a = sv[sl] + tv[sl] + qv[sl]
            gv[sl] = jnp.exp(jnp.maximum(a, 0.01 * a))

        def _scale(e, cc):
            ge = gv[e]
            for j in range(8):
                sl = pl.ds(j * 16, 16)
                rows[e, sl] = rows[e, sl] * ge
            return cc

        lax.fori_loop(0, CHUNK, _scale, 0)
        pltpu.sync_copy(gv, d_sh.at[idx_d], add=True)
        pltpu.sync_copy(rows, h_sh.at[idx_d], add=True)
        return carry

    lax.fori_loop(0, NCHUNK, _chunk, 0)
    plsc.subcore_barrier()

    pltpu.sync_copy(h_sh.at[pl.ds(r0, ROWS_PER_TILE)],
                    hp_hbm.at[c, pl.ds(r0, ROWS_PER_TILE)])
    pltpu.sync_copy(d_sh.at[pl.ds(r0, ROWS_PER_TILE)],
                    dp_hbm.at[c, pl.ds(r0, ROWS_PER_TILE)])


_scmain = pl.kernel(
    _sc_body,
    out_type=[jax.ShapeDtypeStruct((NCORES, NPAD, D), _f32),
              jax.ShapeDtypeStruct((NCORES, NPAD), _f32)],
    mesh=plsc.VectorSubcoreMesh(core_axis_name="c", subcore_axis_name="s",
                                num_cores=NCORES, num_subcores=NSUB),
    scratch_types=[
        pltpu.VMEM_SHARED((NPAD, D), _f32),
        pltpu.VMEM_SHARED((NPAD,), _f32),
        pltpu.VMEM((CHUNK,), jnp.int32),
        pltpu.VMEM((CHUNK,), jnp.int32),
        pltpu.VMEM((CHUNK,), _f32),
        pltpu.VMEM((CHUNK,), _f32),
        pltpu.VMEM((CHUNK,), _f32),
        pltpu.VMEM((CHUNK,), _f32),
        pltpu.VMEM((CHUNK, D), _f32),
        pltpu.VMEM((128, D), _f32),
        pltpu.VMEM((ROWS_PER_TILE,), _f32),
    ],
)


NB = 10
BL = NPAD // NB


def _comb_body(hp_ref, dp_ref, h_ref):
    i = pl.program_id(0)
    den = dp_ref[0, pl.ds(i * BL, BL)] + dp_ref[1, pl.ds(i * BL, BL)]
    den = jnp.where(den == 0.0, 1.0, den)
    h_ref[...] = (hp_ref[0] + hp_ref[1]) / den[:, None]


_combine = pl.pallas_call(
    _comb_body,
    grid=(NB,),
    in_specs=[
        pl.BlockSpec((NCORES, BL, D), lambda i: (0, i, 0)),
        pl.BlockSpec((NCORES, NPAD), lambda i: (0, 0)),
    ],
    out_specs=pl.BlockSpec((BL, D), lambda i: (i, 0)),
    out_shape=jax.ShapeDtypeStruct((NPAD, D), _f32),
)


def kernel(x, edge_index, edge_attr, W_fc, W_rel, b_rel, W_attn):
    src = edge_index[0].astype(jnp.int32)
    dst = edge_index[1].astype(jnp.int32)
    z, st = _prelude(x, W_fc, W_rel, W_attn)
    q3 = _edgeq(edge_attr.reshape(NEB, EBLK, ED), W_rel, W_attn,
                b_rel.reshape(1, D))
    s = st[:, 0]
    t = st[:, 1]
    q = q3.reshape(E)
    hp, dp = _scmain(src, dst, s, t, q, z)
    h = _combine(hp, dp)
    return h[:N]


# speedup vs baseline: 9.0323x; 9.0323x over previous
import functools

import jax
import jax.numpy as jnp
from jax import lax
from jax.experimental import pallas as pl
from jax.experimental.pallas import tpu as pltpu
from jax.experimental.pallas import tpu_sc as plsc

N = 10000
E = 320000
D = 128
ED = 16
NPAD = 10240
NCORES = 2
NSUB = 16
NW = NCORES * NSUB
EPW = E // NW
CHUNK = 80
NCHUNK = EPW // CHUNK
ROWS_PER_TILE = NPAD // NSUB

_f32 = jnp.float32


def _prelude_body(x_ref, wfc_ref, wrel_ref, wattn_ref, z_ref, st_ref):
    z = lax.dot_general(x_ref[...], wfc_ref[...], (((1,), (1,)), ((), ())),
                        precision=lax.Precision.HIGHEST,
                        preferred_element_type=_f32)
    z_ref[...] = z
    wa = wattn_ref[...]
    w1 = wa[:, :D]
    w2 = wa[:, D:]
    u = lax.dot_general(w2, wrel_ref[...], (((1,), (0,)), ((), ())),
                        precision=lax.Precision.HIGHEST,
                        preferred_element_type=_f32)
    uz = u[:, ED:]
    s = lax.dot_general(z, w1, (((1,), (1,)), ((), ())),
                        precision=lax.Precision.HIGHEST,
                        preferred_element_type=_f32)
    t = lax.dot_general(z, uz, (((1,), (1,)), ((), ())),
                        precision=lax.Precision.HIGHEST,
                        preferred_element_type=_f32)
    st_ref[...] = jnp.concatenate([s, t, jnp.zeros((N, 6), _f32)], axis=1)


_prelude = pl.pallas_call(
    _prelude_body,
    out_shape=[jax.ShapeDtypeStruct((N, D), _f32),
               jax.ShapeDtypeStruct((N, 8), _f32)],
)


EBLK = 2000
NEB = E // EBLK


def _q_body(ea_ref, wrel_ref, wattn_ref, brel_ref, q_ref):
    wa = wattn_ref[...]
    w2 = wa[:, D:]
    u = lax.dot_general(w2, wrel_ref[...], (((1,), (0,)), ((), ())),
                        precision=lax.Precision.HIGHEST,
                        preferred_element_type=_f32)
    ue = u[:, :ED]
    c0 = lax.dot_general(brel_ref[...], w2, (((1,), (1,)), ((), ())),
                         precision=lax.Precision.HIGHEST,
                         preferred_element_type=_f32)
    q = lax.dot_general(ue, ea_ref[0], (((1,), (1,)), ((), ())),
                        precision=lax.Precision.HIGHEST,
                        preferred_element_type=_f32)
    q_ref[...] = (q + c0).reshape(1, 1, EBLK)


_edgeq = pl.pallas_call(
    _q_body,
    grid=(NEB,),
    in_specs=[
        pl.BlockSpec((1, EBLK, ED), lambda i: (i, 0, 0)),
        pl.BlockSpec((D, D + ED), lambda i: (0, 0)),
        pl.BlockSpec((1, 2 * D), lambda i: (0, 0)),
        pl.BlockSpec((1, D), lambda i: (0, 0)),
    ],
    out_specs=pl.BlockSpec((1, 1, EBLK), lambda i: (i, 0, 0)),
    out_shape=jax.ShapeDtypeStruct((NEB, 1, EBLK), _f32),
)


def _sc_body(src_hbm, dst_hbm, s_hbm, t_hbm, q_hbm, z_hbm,
             hp_hbm, dp_hbm,
             h_sh, d_sh,
             idx_s, idx_d, sv, tv, qv, gv, rows, zrow, dz):
    c = lax.axis_index("c")
    sid = lax.axis_index("s")

    zeros16 = jnp.zeros((16,), _f32)

    def _zb(i, carry):
        for j in range(8):
            zrow[i, pl.ds(j * 16, 16)] = zeros16
        return carry

    lax.fori_loop(0, 128, _zb, 0)

    def _zd(i, carry):
        dz[pl.ds(i * 16, 16)] = zeros16
        return carry

    lax.fori_loop(0, ROWS_PER_TILE // 16, _zd, 0)

    r0 = sid * ROWS_PER_TILE
    for k in range(ROWS_PER_TILE // 128):
        pltpu.sync_copy(zrow, h_sh.at[pl.ds(r0 + k * 128, 128)])
    pltpu.sync_copy(dz, d_sh.at[pl.ds(r0, ROWS_PER_TILE)])
    plsc.subcore_barrier()

    base = (c * NSUB + sid) * EPW

    def _chunk(i, carry):
        off = base + i * CHUNK
        pltpu.sync_copy(src_hbm.at[pl.ds(off, CHUNK)], idx_s)
        pltpu.sync_copy(dst_hbm.at[pl.ds(off, CHUNK)], idx_d)
        pltpu.sync_copy(q_hbm.at[pl.ds(off, CHUNK)], qv)
        pltpu.sync_copy(s_hbm.at[idx_s], sv)
        pltpu.sync_copy(t_hbm.at[idx_d], tv)
        pltpu.sync_copy(z_hbm.at[idx_s], rows)
        for j in range(CHUNK // 16):
            sl = pl.ds(j * 16, 16)
            a = sv[sl] + tv[sl] + qv[sl]
            gv[sl] = jnp.exp(jnp.maximum(a, 0.01 * a))

        def _scale(k, cc):
            g16 = gv[pl.ds(k * 16, 16)]
            for j in range(16):
                ge = g16[j]
                e = k * 16 + j
                for i in range(8):
                    sl = pl.ds(i * 16, 16)
                    rows[e, sl] = rows[e, sl] * ge
            return cc

        lax.fori_loop(0, CHUNK // 16, _scale, 0)
        pltpu.sync_copy(gv, d_sh.at[idx_d], add=True)
        pltpu.sync_copy(rows, h_sh.at[idx_d], add=True)
        return carry

    lax.fori_loop(0, NCHUNK, _chunk, 0)
    plsc.subcore_barrier()

    pltpu.sync_copy(h_sh.at[pl.ds(r0, ROWS_PER_TILE)],
                    hp_hbm.at[c, pl.ds(r0, ROWS_PER_TILE)])
    pltpu.sync_copy(d_sh.at[pl.ds(r0, ROWS_PER_TILE)],
                    dp_hbm.at[c, pl.ds(r0, ROWS_PER_TILE)])


_scmain = pl.kernel(
    _sc_body,
    out_type=[jax.ShapeDtypeStruct((NCORES, NPAD, D), _f32),
              jax.ShapeDtypeStruct((NCORES, NPAD), _f32)],
    mesh=plsc.VectorSubcoreMesh(core_axis_name="c", subcore_axis_name="s",
                                num_cores=NCORES, num_subcores=NSUB),
    scratch_types=[
        pltpu.VMEM_SHARED((NPAD, D), _f32),
        pltpu.VMEM_SHARED((NPAD,), _f32),
        pltpu.VMEM((CHUNK,), jnp.int32),
        pltpu.VMEM((CHUNK,), jnp.int32),
        pltpu.VMEM((CHUNK,), _f32),
        pltpu.VMEM((CHUNK,), _f32),
        pltpu.VMEM((CHUNK,), _f32),
        pltpu.VMEM((CHUNK,), _f32),
        pltpu.VMEM((CHUNK, D), _f32),
        pltpu.VMEM((128, D), _f32),
        pltpu.VMEM((ROWS_PER_TILE,), _f32),
    ],
)


NB = 10
BL = NPAD // NB


def _comb_body(hp_ref, dp_ref, h_ref):
    i = pl.program_id(0)
    den = dp_ref[0, pl.ds(i * BL, BL)] + dp_ref[1, pl.ds(i * BL, BL)]
    den = jnp.where(den == 0.0, 1.0, den)
    h_ref[...] = (hp_ref[0] + hp_ref[1]) / den[:, None]


_combine = pl.pallas_call(
    _comb_body,
    grid=(NB,),
    in_specs=[
        pl.BlockSpec((NCORES, BL, D), lambda i: (0, i, 0)),
        pl.BlockSpec((NCORES, NPAD), lambda i: (0, 0)),
    ],
    out_specs=pl.BlockSpec((BL, D), lambda i: (i, 0)),
    out_shape=jax.ShapeDtypeStruct((NPAD, D), _f32),
)


def kernel(x, edge_index, edge_attr, W_fc, W_rel, b_rel, W_attn):
    src = edge_index[0].astype(jnp.int32)
    dst = edge_index[1].astype(jnp.int32)
    z, st = _prelude(x, W_fc, W_rel, W_attn)
    q3 = _edgeq(edge_attr.reshape(NEB, EBLK, ED), W_rel, W_attn,
                b_rel.reshape(1, D))
    s = st[:, 0]
    t = st[:, 1]
    q = q3.reshape(E)
    hp, dp = _scmain(src, dst, s, t, q, z)
    h = _combine(hp, dp)
    return h[:N]


# speedup vs baseline: 17.6086x; 1.9495x over previous
import functools

import jax
import jax.numpy as jnp
from jax import lax
from jax.experimental import pallas as pl
from jax.experimental.pallas import tpu as pltpu
from jax.experimental.pallas import tpu_sc as plsc

N = 10000
E = 320000
D = 128
ED = 16
NPAD = 10240
NCORES = 2
NSUB = 16
NW = NCORES * NSUB
EPW = E // NW
CHUNK = 80
NCHUNK = EPW // CHUNK
ROWS_PER_TILE = NPAD // NSUB

_f32 = jnp.float32


def _prelude_body(x_ref, wfc_ref, wrel_ref, wattn_ref, z_ref, st_ref):
    z = lax.dot_general(x_ref[...], wfc_ref[...], (((1,), (1,)), ((), ())),
                        precision=lax.Precision.HIGHEST,
                        preferred_element_type=_f32)
    z_ref[...] = z
    wa = wattn_ref[...]
    w1 = wa[:, :D]
    w2 = wa[:, D:]
    u = lax.dot_general(w2, wrel_ref[...], (((1,), (0,)), ((), ())),
                        precision=lax.Precision.HIGHEST,
                        preferred_element_type=_f32)
    uz = u[:, ED:]
    s = lax.dot_general(z, w1, (((1,), (1,)), ((), ())),
                        precision=lax.Precision.HIGHEST,
                        preferred_element_type=_f32)
    t = lax.dot_general(z, uz, (((1,), (1,)), ((), ())),
                        precision=lax.Precision.HIGHEST,
                        preferred_element_type=_f32)
    st_ref[...] = jnp.concatenate([s, t, jnp.zeros((N, 6), _f32)], axis=1)


_prelude = pl.pallas_call(
    _prelude_body,
    out_shape=[jax.ShapeDtypeStruct((N, D), _f32),
               jax.ShapeDtypeStruct((N, 8), _f32)],
)


EBLK = 2000
NEB = E // EBLK


def _q_body(ea_ref, wrel_ref, wattn_ref, brel_ref, src_ref, dst_ref,
            q_ref, pk_ref):
    wa = wattn_ref[...]
    w2 = wa[:, D:]
    u = lax.dot_general(w2, wrel_ref[...], (((1,), (0,)), ((), ())),
                        precision=lax.Precision.HIGHEST,
                        preferred_element_type=_f32)
    ue = u[:, :ED]
    c0 = lax.dot_general(brel_ref[...], w2, (((1,), (1,)), ((), ())),
                         precision=lax.Precision.HIGHEST,
                         preferred_element_type=_f32)
    q = lax.dot_general(ue, ea_ref[0], (((1,), (1,)), ((), ())),
                        precision=lax.Precision.HIGHEST,
                        preferred_element_type=_f32)
    q_ref[...] = (q + c0).reshape(1, 1, EBLK)
    pk_ref[...] = src_ref[...] + dst_ref[...] * 16384


_edgeq = pl.pallas_call(
    _q_body,
    grid=(NEB,),
    in_specs=[
        pl.BlockSpec((1, EBLK, ED), lambda i: (i, 0, 0)),
        pl.BlockSpec((D, D + ED), lambda i: (0, 0)),
        pl.BlockSpec((1, 2 * D), lambda i: (0, 0)),
        pl.BlockSpec((1, D), lambda i: (0, 0)),
        pl.BlockSpec((1, 1, EBLK), lambda i: (i, 0, 0)),
        pl.BlockSpec((1, 1, EBLK), lambda i: (i, 0, 0)),
    ],
    out_specs=[pl.BlockSpec((1, 1, EBLK), lambda i: (i, 0, 0)),
               pl.BlockSpec((1, 1, EBLK), lambda i: (i, 0, 0))],
    out_shape=[jax.ShapeDtypeStruct((NEB, 1, EBLK), _f32),
               jax.ShapeDtypeStruct((NEB, 1, EBLK), jnp.int32)],
)


def _sc_body(pk_hbm, s_hbm, t_hbm, q_hbm, z_hbm, z2d_hbm, z1d_hbm,
             hp_hbm, dp_hbm,
             h_sh, d_sh,
             sem_lin, semg0, semg1, semsc0, semsc1):
    def _scoped(ipa, qa,
                isb0, idb0, sv0, tv0, gv0, rows0,
                isb1, idb1, sv1, tv1, gv1, rows1):
        _sc_inner(pk_hbm, s_hbm, t_hbm, q_hbm, z_hbm, z2d_hbm,
                  z1d_hbm, hp_hbm, dp_hbm, h_sh, d_sh, ipa, qa,
                  isb0, idb0, sv0, tv0, gv0, rows0,
                  isb1, idb1, sv1, tv1, gv1, rows1,
                  sem_lin, semg0, semg1, semsc0, semsc1)

    pl.run_scoped(
        _scoped,
        pltpu.VMEM((EPW,), jnp.int32),
        pltpu.VMEM((EPW,), _f32),
        pltpu.VMEM((CHUNK,), jnp.int32),
        pltpu.VMEM((CHUNK,), jnp.int32),
        pltpu.VMEM((CHUNK,), _f32),
        pltpu.VMEM((CHUNK,), _f32),
        pltpu.VMEM((CHUNK,), _f32),
        pltpu.VMEM((CHUNK, D), _f32),
        pltpu.VMEM((CHUNK,), jnp.int32),
        pltpu.VMEM((CHUNK,), jnp.int32),
        pltpu.VMEM((CHUNK,), _f32),
        pltpu.VMEM((CHUNK,), _f32),
        pltpu.VMEM((CHUNK,), _f32),
        pltpu.VMEM((CHUNK, D), _f32),
    )


def _sc_inner(pk_hbm, s_hbm, t_hbm, q_hbm, z_hbm, z2d_hbm, z1d_hbm,
              hp_hbm, dp_hbm, h_sh, d_sh, ipa, qa,
              isb0, idb0, sv0, tv0, gv0, rows0,
              isb1, idb1, sv1, tv1, gv1, rows1,
              sem_lin, semg0, semg1, semsc0, semsc1):
    c = lax.axis_index("c")
    sid = lax.axis_index("s")
    w = c * NSUB + sid

    e0 = w * EPW
    dl0 = pltpu.async_copy(pk_hbm.at[pl.ds(e0, EPW)], ipa, sem_lin)
    dl2 = pltpu.async_copy(q_hbm.at[pl.ds(e0, EPW)], qa, sem_lin)

    r0 = sid * ROWS_PER_TILE
    pltpu.sync_copy(z2d_hbm, h_sh.at[pl.ds(r0, ROWS_PER_TILE)])
    pltpu.sync_copy(z1d_hbm, d_sh.at[pl.ds(r0, ROWS_PER_TILE)])
    dl0.wait()
    dl2.wait()
    plsc.subcore_barrier()

    bufs = ((isb0, idb0, sv0, tv0, gv0, rows0, semg0, semsc0),
            (isb1, idb1, sv1, tv1, gv1, rows1, semg1, semsc1))

    def _stage_idx(i, b):
        isb, idb = bufs[b][0], bufs[b][1]
        for j in range(CHUNK // 16):
            sl = pl.ds(j * 16, 16)
            u = ipa[pl.ds(i * CHUNK + j * 16, 16)]
            isb[sl] = jnp.bitwise_and(u, 16383)
            idb[sl] = lax.shift_right_logical(u, 14)

    def _issue_gathers(i, b):
        isb, idb, sv, tv, gv, rows, semg, semsc = bufs[b]
        pltpu.async_copy(s_hbm.at[isb], sv, semg)
        pltpu.async_copy(t_hbm.at[idb], tv, semg)
        pltpu.async_copy(z_hbm.at[isb], rows, semg)

    def _wait_gathers(i, b):
        isb, idb, sv, tv, gv, rows, semg, semsc = bufs[b]
        pltpu.make_async_copy(s_hbm.at[isb], sv, semg).wait()
        pltpu.make_async_copy(t_hbm.at[idb], tv, semg).wait()
        pltpu.make_async_copy(z_hbm.at[isb], rows, semg).wait()

    def _issue_scatters(i, b):
        isb, idb, sv, tv, gv, rows, semg, semsc = bufs[b]
        pltpu.async_copy(gv, d_sh.at[idb], semsc, add=True)
        pltpu.async_copy(rows, h_sh.at[idb], semsc, add=True)

    def _wait_scatters(i, b):
        isb, idb, sv, tv, gv, rows, semg, semsc = bufs[b]
        pltpu.make_async_copy(gv, d_sh.at[idb], semsc).wait()
        pltpu.make_async_copy(rows, h_sh.at[idb], semsc).wait()

    def _compute(i, b):
        isb, idb, sv, tv, gv, rows, semg, semsc = bufs[b]
        for j in range(CHUNK // 16):
            sl = pl.ds(j * 16, 16)
            a = sv[sl] + tv[sl] + qa[pl.ds(i * CHUNK + j * 16, 16)]
            gv[sl] = jnp.exp(jnp.maximum(a, 0.01 * a))

        def _scale(k, cc):
            g16 = gv[pl.ds(k * 16, 16)]
            for j in range(16):
                ge = g16[j]
                e = k * 16 + j
                for jj in range(8):
                    sl = pl.ds(jj * 16, 16)
                    rows[e, sl] = rows[e, sl] * ge
            return cc

        lax.fori_loop(0, CHUNK // 16, _scale, 0)

    _stage_idx(0, 0)
    _issue_gathers(0, 0)

    def _pair(k, carry):
        for b in range(2):
            i = 2 * k + b
            nxt = 1 - b

            @pl.when(i < NCHUNK)
            def _():
                @pl.when(i + 1 < NCHUNK)
                def _():
                    @pl.when(i >= 1)
                    def _():
                        _wait_scatters(i - 1, nxt)

                    _stage_idx(i + 1, nxt)
                    _issue_gathers(i + 1, nxt)

                _wait_gathers(i, b)
                _compute(i, b)
                _issue_scatters(i, b)

        return carry

    lax.fori_loop(0, (NCHUNK + 1) // 2, _pair, 0)
    _wait_scatters(NCHUNK - 2, NCHUNK % 2)
    _wait_scatters(NCHUNK - 1, (NCHUNK - 1) % 2)
    plsc.subcore_barrier()

    pltpu.sync_copy(h_sh.at[pl.ds(r0, ROWS_PER_TILE)],
                    hp_hbm.at[c, pl.ds(r0, ROWS_PER_TILE)])
    pltpu.sync_copy(d_sh.at[pl.ds(r0, ROWS_PER_TILE)],
                    dp_hbm.at[c, pl.ds(r0, ROWS_PER_TILE)])


_scmain = pl.kernel(
    _sc_body,
    out_type=[jax.ShapeDtypeStruct((NCORES, NPAD, D), _f32),
              jax.ShapeDtypeStruct((NCORES, NPAD), _f32)],
    mesh=plsc.VectorSubcoreMesh(core_axis_name="c", subcore_axis_name="s",
                                num_cores=NCORES, num_subcores=NSUB),
    scratch_types=[
        pltpu.VMEM_SHARED((NPAD, D), _f32),
        pltpu.VMEM_SHARED((NPAD,), _f32),
        pltpu.SemaphoreType.DMA,
        pltpu.SemaphoreType.DMA,
        pltpu.SemaphoreType.DMA,
        pltpu.SemaphoreType.DMA,
        pltpu.SemaphoreType.DMA,
    ],
)


NB = 10
BL = NPAD // NB


def _comb_body(hp_ref, dp_ref, h_ref):
    i = pl.program_id(0)
    den = dp_ref[0, pl.ds(i * BL, BL)] + dp_ref[1, pl.ds(i * BL, BL)]
    den = jnp.where(den == 0.0, 1.0, den)
    h_ref[...] = (hp_ref[0] + hp_ref[1]) / den[:, None]


_combine = pl.pallas_call(
    _comb_body,
    grid=(NB,),
    in_specs=[
        pl.BlockSpec((NCORES, BL, D), lambda i: (0, i, 0)),
        pl.BlockSpec((NCORES, NPAD), lambda i: (0, 0)),
    ],
    out_specs=pl.BlockSpec((BL, D), lambda i: (i, 0)),
    out_shape=jax.ShapeDtypeStruct((NPAD, D), _f32),
)


def kernel(x, edge_index, edge_attr, W_fc, W_rel, b_rel, W_attn):
    src = edge_index[0].astype(jnp.int32)
    dst = edge_index[1].astype(jnp.int32)
    z, st = _prelude(x, W_fc, W_rel, W_attn)
    q3, pk3 = _edgeq(edge_attr.reshape(NEB, EBLK, ED), W_rel, W_attn,
                     b_rel.reshape(1, D),
                     src.reshape(NEB, 1, EBLK), dst.reshape(NEB, 1, EBLK))
    s = st[:, 0]
    t = st[:, 1]
    hp, dp = _scmain(pk3.reshape(E), s, t, q3.reshape(E), z,
                     jnp.zeros((ROWS_PER_TILE, D), _f32),
                     jnp.zeros((ROWS_PER_TILE,), _f32))
    h = _combine(hp, dp)
    return h[:N]


# speedup vs baseline: 20.5067x; 1.1646x over previous
import functools

import jax
import jax.numpy as jnp
from jax import lax
from jax.experimental import pallas as pl
from jax.experimental.pallas import tpu as pltpu
from jax.experimental.pallas import tpu_sc as plsc

N = 10000
E = 320000
D = 128
ED = 16
NPAD = 10240
NCORES = 2
NSUB = 16
NW = NCORES * NSUB
EPW = E // NW
CHUNK = 80
NCHUNK = EPW // CHUNK
ROWS_PER_TILE = NPAD // NSUB

_f32 = jnp.float32


BN = 1000
NBN = N // BN


def _prelude_body(x_ref, wfc_ref, wrel_ref, wattn_ref, z_ref, st_ref):
    z = lax.dot_general(x_ref[...], wfc_ref[...], (((1,), (1,)), ((), ())),
                        precision=lax.Precision.HIGHEST,
                        preferred_element_type=_f32)
    z_ref[...] = z
    wa = wattn_ref[...]
    w1 = wa[:, :D]
    w2 = wa[:, D:]
    u = lax.dot_general(w2, wrel_ref[...], (((1,), (0,)), ((), ())),
                        precision=lax.Precision.HIGHEST,
                        preferred_element_type=_f32)
    uz = u[:, ED:]
    s = jnp.sum(z * w1, axis=1, keepdims=True)
    t = jnp.sum(z * uz, axis=1, keepdims=True)
    st_ref[...] = jnp.concatenate([s, t, jnp.zeros((BN, 6), _f32)], axis=1)


_prelude = pl.pallas_call(
    _prelude_body,
    grid=(NBN,),
    in_specs=[
        pl.BlockSpec((BN, D), lambda i: (i, 0)),
        pl.BlockSpec((D, D), lambda i: (0, 0)),
        pl.BlockSpec((D, D + ED), lambda i: (0, 0)),
        pl.BlockSpec((1, 2 * D), lambda i: (0, 0)),
    ],
    out_specs=[pl.BlockSpec((BN, D), lambda i: (i, 0)),
               pl.BlockSpec((BN, 8), lambda i: (i, 0))],
    out_shape=[jax.ShapeDtypeStruct((N, D), _f32),
               jax.ShapeDtypeStruct((N, 8), _f32)],
)


EBLK = 8000
NEB = E // EBLK


def _q_body(ea_ref, wrel_ref, wattn_ref, brel_ref, src_ref, dst_ref,
            q_ref, pk_ref):
    wa = wattn_ref[...]
    w2 = wa[:, D:]
    u = lax.dot_general(w2, wrel_ref[...], (((1,), (0,)), ((), ())),
                        precision=lax.Precision.HIGHEST,
                        preferred_element_type=_f32)
    ue = u[:, :ED]
    c0 = lax.dot_general(brel_ref[...], w2, (((1,), (1,)), ((), ())),
                         precision=lax.Precision.HIGHEST,
                         preferred_element_type=_f32)
    q = lax.dot_general(ue, ea_ref[0], (((1,), (1,)), ((), ())),
                        precision=lax.Precision.HIGHEST,
                        preferred_element_type=_f32)
    q_ref[...] = (q + c0).reshape(1, 1, EBLK)
    pk_ref[...] = (src_ref[...] + dst_ref[...] * 16384).reshape(1, 1, EBLK)


_edgeq = pl.pallas_call(
    _q_body,
    grid=(NEB,),
    in_specs=[
        pl.BlockSpec((1, EBLK, ED), lambda i: (i, 0, 0)),
        pl.BlockSpec((D, D + ED), lambda i: (0, 0)),
        pl.BlockSpec((1, 2 * D), lambda i: (0, 0)),
        pl.BlockSpec((1, D), lambda i: (0, 0)),
        pl.BlockSpec((1, 1, 1, EBLK), lambda i: (0, i, 0, 0)),
        pl.BlockSpec((1, 1, 1, EBLK), lambda i: (1, i, 0, 0)),
    ],
    out_specs=[pl.BlockSpec((1, 1, EBLK), lambda i: (i, 0, 0)),
               pl.BlockSpec((1, 1, EBLK), lambda i: (i, 0, 0))],
    out_shape=[jax.ShapeDtypeStruct((NEB, 1, EBLK), _f32),
               jax.ShapeDtypeStruct((NEB, 1, EBLK), jnp.int32)],
)


def _sc_body(pk_hbm, s_hbm, t_hbm, q_hbm, z_hbm, z2d_hbm, z1d_hbm,
             hp_hbm, dp_hbm,
             h_sh, d_sh,
             sem_lin, semg0, semg1, semsc0, semsc1):
    def _scoped(ipa, qa,
                isb0, idb0, sv0, tv0, gv0, rows0,
                isb1, idb1, sv1, tv1, gv1, rows1):
        _sc_inner(pk_hbm, s_hbm, t_hbm, q_hbm, z_hbm, z2d_hbm,
                  z1d_hbm, hp_hbm, dp_hbm, h_sh, d_sh, ipa, qa,
                  isb0, idb0, sv0, tv0, gv0, rows0,
                  isb1, idb1, sv1, tv1, gv1, rows1,
                  sem_lin, semg0, semg1, semsc0, semsc1)

    pl.run_scoped(
        _scoped,
        pltpu.VMEM((EPW,), jnp.int32),
        pltpu.VMEM((EPW,), _f32),
        pltpu.VMEM((CHUNK,), jnp.int32),
        pltpu.VMEM((CHUNK,), jnp.int32),
        pltpu.VMEM((CHUNK,), _f32),
        pltpu.VMEM((CHUNK,), _f32),
        pltpu.VMEM((CHUNK,), _f32),
        pltpu.VMEM((CHUNK, D), _f32),
        pltpu.VMEM((CHUNK,), jnp.int32),
        pltpu.VMEM((CHUNK,), jnp.int32),
        pltpu.VMEM((CHUNK,), _f32),
        pltpu.VMEM((CHUNK,), _f32),
        pltpu.VMEM((CHUNK,), _f32),
        pltpu.VMEM((CHUNK, D), _f32),
    )


def _sc_inner(pk_hbm, s_hbm, t_hbm, q_hbm, z_hbm, z2d_hbm, z1d_hbm,
              hp_hbm, dp_hbm, h_sh, d_sh, ipa, qa,
              isb0, idb0, sv0, tv0, gv0, rows0,
              isb1, idb1, sv1, tv1, gv1, rows1,
              sem_lin, semg0, semg1, semsc0, semsc1):
    c = lax.axis_index("c")
    sid = lax.axis_index("s")
    w = c * NSUB + sid

    e0 = w * EPW
    dl0 = pltpu.async_copy(pk_hbm.at[pl.ds(e0, EPW)], ipa, sem_lin)
    dl2 = pltpu.async_copy(q_hbm.at[pl.ds(e0, EPW)], qa, sem_lin)

    r0 = sid * ROWS_PER_TILE
    pltpu.sync_copy(z2d_hbm, h_sh.at[pl.ds(r0, ROWS_PER_TILE)])
    pltpu.sync_copy(z1d_hbm, d_sh.at[pl.ds(r0, ROWS_PER_TILE)])
    dl0.wait()
    dl2.wait()
    plsc.subcore_barrier()

    bufs = ((isb0, idb0, sv0, tv0, gv0, rows0, semg0, semsc0),
            (isb1, idb1, sv1, tv1, gv1, rows1, semg1, semsc1))

    def _stage_idx(i, b):
        isb, idb = bufs[b][0], bufs[b][1]
        for j in range(CHUNK // 16):
            sl = pl.ds(j * 16, 16)
            u = ipa[pl.ds(i * CHUNK + j * 16, 16)]
            isb[sl] = jnp.bitwise_and(u, 16383)
            idb[sl] = lax.shift_right_logical(u, 14)

    def _issue_gathers(i, b):
        isb, idb, sv, tv, gv, rows, semg, semsc = bufs[b]
        pltpu.async_copy(s_hbm.at[isb], sv, semg)
        pltpu.async_copy(t_hbm.at[idb], tv, semg)
        pltpu.async_copy(z_hbm.at[isb], rows, semg)

    def _wait_gathers(i, b):
        isb, idb, sv, tv, gv, rows, semg, semsc = bufs[b]
        pltpu.make_async_copy(s_hbm.at[isb], sv, semg).wait()
        pltpu.make_async_copy(t_hbm.at[idb], tv, semg).wait()
        pltpu.make_async_copy(z_hbm.at[isb], rows, semg).wait()

    def _issue_scatters(i, b):
        isb, idb, sv, tv, gv, rows, semg, semsc = bufs[b]
        pltpu.async_copy(gv, d_sh.at[idb], semsc, add=True)
        pltpu.async_copy(rows, h_sh.at[idb], semsc, add=True)

    def _wait_scatters(i, b):
        isb, idb, sv, tv, gv, rows, semg, semsc = bufs[b]
        pltpu.make_async_copy(gv, d_sh.at[idb], semsc).wait()
        pltpu.make_async_copy(rows, h_sh.at[idb], semsc).wait()

    def _compute(i, b):
        isb, idb, sv, tv, gv, rows, semg, semsc = bufs[b]
        for j in range(CHUNK // 16):
            sl = pl.ds(j * 16, 16)
            a = sv[sl] + tv[sl] + qa[pl.ds(i * CHUNK + j * 16, 16)]
            gv[sl] = jnp.exp(jnp.maximum(a, 0.01 * a))

        def _scale(k, cc):
            g16 = gv[pl.ds(k * 16, 16)]
            for j in range(16):
                ge = g16[j]
                e = k * 16 + j
                for jj in range(8):
                    sl = pl.ds(jj * 16, 16)
                    rows[e, sl] = rows[e, sl] * ge
            return cc

        lax.fori_loop(0, CHUNK // 16, _scale, 0)

    _stage_idx(0, 0)
    _issue_gathers(0, 0)

    def _pair(k, carry):
        for b in range(2):
            i = 2 * k + b
            nxt = 1 - b

            @pl.when(i < NCHUNK)
            def _():
                @pl.when(i + 1 < NCHUNK)
                def _():
                    @pl.when(i >= 1)
                    def _():
                        _wait_scatters(i - 1, nxt)

                    _stage_idx(i + 1, nxt)
                    _issue_gathers(i + 1, nxt)

                _wait_gathers(i, b)
                _compute(i, b)
                _issue_scatters(i, b)

        return carry

    lax.fori_loop(0, (NCHUNK + 1) // 2, _pair, 0)
    _wait_scatters(NCHUNK - 2, NCHUNK % 2)
    _wait_scatters(NCHUNK - 1, (NCHUNK - 1) % 2)
    plsc.subcore_barrier()

    pltpu.sync_copy(h_sh.at[pl.ds(r0, ROWS_PER_TILE)],
                    hp_hbm.at[c, pl.ds(r0, ROWS_PER_TILE)])
    pltpu.sync_copy(d_sh.at[pl.ds(r0, ROWS_PER_TILE)],
                    dp_hbm.at[c, pl.ds(r0, ROWS_PER_TILE)])


_scmain = pl.kernel(
    _sc_body,
    out_type=[jax.ShapeDtypeStruct((NCORES, NPAD, D), _f32),
              jax.ShapeDtypeStruct((NCORES, NPAD), _f32)],
    mesh=plsc.VectorSubcoreMesh(core_axis_name="c", subcore_axis_name="s",
                                num_cores=NCORES, num_subcores=NSUB),
    scratch_types=[
        pltpu.VMEM_SHARED((NPAD, D), _f32),
        pltpu.VMEM_SHARED((NPAD,), _f32),
        pltpu.SemaphoreType.DMA,
        pltpu.SemaphoreType.DMA,
        pltpu.SemaphoreType.DMA,
        pltpu.SemaphoreType.DMA,
        pltpu.SemaphoreType.DMA,
    ],
)


NB = 10
BL = NPAD // NB


def _comb_body(hp_ref, dp_ref, h_ref):
    i = pl.program_id(0)
    den = dp_ref[0, pl.ds(i * BL, BL)] + dp_ref[1, pl.ds(i * BL, BL)]
    den = jnp.where(den == 0.0, 1.0, den)
    h_ref[...] = (hp_ref[0] + hp_ref[1]) / den[:, None]


_combine = pl.pallas_call(
    _comb_body,
    grid=(NB,),
    in_specs=[
        pl.BlockSpec((NCORES, BL, D), lambda i: (0, i, 0)),
        pl.BlockSpec((NCORES, NPAD), lambda i: (0, 0)),
    ],
    out_specs=pl.BlockSpec((BL, D), lambda i: (i, 0)),
    out_shape=jax.ShapeDtypeStruct((NPAD, D), _f32),
)


def kernel(x, edge_index, edge_attr, W_fc, W_rel, b_rel, W_attn):
    ei3 = edge_index.astype(jnp.int32).reshape(2, NEB, 1, EBLK)
    z, st = _prelude(x, W_fc, W_rel, W_attn)
    q3, pk3 = _edgeq(edge_attr.reshape(NEB, EBLK, ED), W_rel, W_attn,
                     b_rel.reshape(1, D), ei3, ei3)
    s = st[:, 0]
    t = st[:, 1]
    hp, dp = _scmain(pk3.reshape(E), s, t, q3.reshape(E), z,
                     jnp.zeros((ROWS_PER_TILE, D), _f32),
                     jnp.zeros((ROWS_PER_TILE,), _f32))
    h = _combine(hp, dp)
    return h[:N]
